# baseline (speedup 1.0000x reference)
import jax
import jax.numpy as jnp
from jax import lax
from jax.experimental import pallas as pl
from jax.experimental.pallas import tpu as pltpu
from jax.experimental.pallas import tpu_sc as plsc

VOCAB_TOTAL = 1000000
BATCH = 4096
SEQ = 200
D = 64
DPAD = 128
NC = 2
NS = 16
NW = NC * NS
WB = BATCH // NW
NBUF = 5
AHEAD = 3
GROUPS = SEQ // NBUF

def _emb_body(xt_hbm, tbl_hbm, out_hbm, idx_v, rows_v, *sems):
    gsems, osems = sems[:NBUF], sems[NBUF:]
    wid = lax.axis_index("s") * NC + lax.axis_index("c")
    bbase = wid * WB
    pltpu.sync_copy(xt_hbm.at[:, pl.ds(bbase, WB)], idx_v)

    def fire_gather(s, slot):
        pltpu.async_copy(
            tbl_hbm.at[idx_v.at[s]],
            rows_v.at[pl.ds(slot * WB, WB)],
            gsems[slot])

    def wait_gather(slot):
        pltpu.make_async_copy(
            tbl_hbm.at[idx_v.at[0]],
            rows_v.at[pl.ds(slot * WB, WB)],
            gsems[slot]).wait()

    def fire_scatter(s, slot):
        pltpu.async_copy(
            rows_v.at[pl.ds(slot * WB, WB)],
            out_hbm.at[s, pl.ds(bbase, WB), :],
            osems[slot])

    def wait_scatter(slot):
        pltpu.make_async_copy(
            rows_v.at[pl.ds(slot * WB, WB)],
            out_hbm.at[0, pl.ds(bbase, WB), :],
            osems[slot]).wait()

    def process(s, slot, wait_scat, fire_ahead):
        wait_gather(slot)
        fire_scatter(s, slot)
        if fire_ahead:
            slot2 = (slot + AHEAD) % NBUF
            if wait_scat:
                wait_scatter(slot2)
            fire_gather(s + AHEAD, slot2)

    for s in range(AHEAD):
        fire_gather(s, s % NBUF)

    for b in range(NBUF):
        process(b, b, wait_scat=(b + AHEAD >= NBUF), fire_ahead=True)

    def group(m, carry):
        s0 = m * NBUF
        for b in range(NBUF):
            process(s0 + b, b, wait_scat=True, fire_ahead=True)
        return carry

    lax.fori_loop(1, GROUPS - 1, group, 0)

    s0 = (GROUPS - 1) * NBUF
    for b in range(NBUF):
        process(s0 + b, b, wait_scat=True, fire_ahead=(b + AHEAD < NBUF))
    for b in range(NBUF):
        wait_scatter(b)


@jax.jit
def _emb_call(xt, tbl):
    mesh = plsc.VectorSubcoreMesh(core_axis_name="c", subcore_axis_name="s")
    return pl.kernel(
        _emb_body,
        mesh=mesh,
        out_type=jax.ShapeDtypeStruct((SEQ, BATCH, DPAD), jnp.float32),
        scratch_types=[
            pltpu.VMEM((SEQ, WB), jnp.int32),
            pltpu.VMEM((NBUF * WB, DPAD), jnp.float32),
        ] + [pltpu.SemaphoreType.DMA] * (2 * NBUF),
        compiler_params=pltpu.CompilerParams(use_tc_tiling_on_sc=True),
    )(xt, tbl)


def kernel(x, table):
    xt = jnp.transpose(x)
    tbl = jnp.pad(table, ((0, 0), (0, DPAD - D)))
    return _emb_call(xt, tbl)[:, :, :D]
